# baseline (speedup 1.0000x reference)
import jax
import jax.numpy as jnp
from jax import lax
from jax.experimental import pallas as pl
from jax.experimental.pallas import tpu as pltpu
from jax.experimental.pallas import tpu_sc as plsc

N_BATCH = 4
SEQ = 4096
EMB = 2048
NC, NS = 2, 16
NW = NC * NS
S_PER_W = SEQ // NW
C = 16
N_CHUNKS = S_PER_W // C
CHUNK_W = C * EMB
VECS_PER_CHUNK = CHUNK_W // 16


def _sc_body(x_hbm, pos_hbm, out_hbm, bufp, bufx0, bufx1, sem0, sem1):
    wid = lax.axis_index("s") * NC + lax.axis_index("c")
    s_base = wid * S_PER_W

    def chunk_body(g, _):
        s0 = s_base + g * C
        p_off = s0 * EMB
        pltpu.sync_copy(pos_hbm.at[pl.ds(p_off, CHUNK_W)], bufp)

        cp0 = pltpu.async_copy(
            x_hbm.at[pl.ds(0 * SEQ * EMB + p_off, CHUNK_W)], bufx0, sem0
        )

        def do_batch(b, bufx, cp, nxt):
            cp.wait()
            nxt_cp = None
            if nxt is not None:
                nxt_b, nxt_buf, nxt_sem = nxt
                nxt_cp = pltpu.async_copy(
                    x_hbm.at[pl.ds(nxt_b * SEQ * EMB + p_off, CHUNK_W)],
                    nxt_buf,
                    nxt_sem,
                )

            def vec_add(i, _):
                sl = pl.ds(i * 16, 16)
                bufx[sl] = bufx[sl] + bufp[sl]
                return 0

            lax.fori_loop(0, VECS_PER_CHUNK, vec_add, 0, unroll=8)
            pltpu.sync_copy(bufx, out_hbm.at[pl.ds(b * SEQ * EMB + p_off, CHUNK_W)])
            return nxt_cp

        cp1 = do_batch(0, bufx0, cp0, (1, bufx1, sem1))
        cp2 = do_batch(1, bufx1, cp1, (2, bufx0, sem0))
        cp3 = do_batch(2, bufx0, cp2, (3, bufx1, sem1))
        do_batch(3, bufx1, cp3, None)
        return 0

    lax.fori_loop(0, N_CHUNKS, chunk_body, 0)


def kernel(x, pos_embedding):
    k = pl.kernel(
        _sc_body,
        mesh=plsc.VectorSubcoreMesh(core_axis_name="c", subcore_axis_name="s"),
        out_type=jax.ShapeDtypeStruct((N_BATCH * SEQ * EMB,), jnp.float32),
        scratch_types=[
            pltpu.VMEM((CHUNK_W,), jnp.float32),
            pltpu.VMEM((CHUNK_W,), jnp.float32),
            pltpu.VMEM((CHUNK_W,), jnp.float32),
            pltpu.SemaphoreType.DMA,
            pltpu.SemaphoreType.DMA,
        ],
    )
    out = k(x.reshape(-1), pos_embedding.reshape(-1))
    return out.reshape(N_BATCH, SEQ, EMB)


# speedup vs baseline: 1.5097x; 1.5097x over previous
import jax
import jax.numpy as jnp
from jax import lax
from jax.experimental import pallas as pl
from jax.experimental.pallas import tpu as pltpu
from jax.experimental.pallas import tpu_sc as plsc

N_BATCH = 4
SEQ = 4096
EMB = 2048
NC, NS = 2, 16
NW = NC * NS
S_PER_W = SEQ // NW
C = 8
N_CHUNKS = S_PER_W // C
CHUNK_W = C * EMB
VECS = CHUNK_W // 16
IMG_W = SEQ * EMB


def _sc_body(x_hbm, pos_hbm, out_hbm,
             bufp0, bufp1, bx0, bx1, bx2, bx3,
             psem0, psem1, lsem0, lsem1, lsem2, lsem3,
             ssem0, ssem1, ssem2, ssem3):
    wid = lax.axis_index("s") * NC + lax.axis_index("c")
    p_base = wid * S_PER_W * EMB

    bufp = (bufp0, bufp1)
    bx = (bx0, bx1, bx2, bx3)
    lsem = (lsem0, lsem1, lsem2, lsem3)
    ssem = (ssem0, ssem1, ssem2, ssem3)

    pltpu.async_copy(pos_hbm.at[pl.ds(p_base, CHUNK_W)], bufp0, psem0)

    def do_chunk(g, p):
        off = p_base + g * CHUNK_W

        pltpu.make_async_copy(
            pos_hbm.at[pl.ds(0, CHUNK_W)], bufp[p], (psem0, psem1)[p]
        ).wait()

        @pl.when(g + 1 < N_CHUNKS)
        def _():
            pltpu.async_copy(
                pos_hbm.at[pl.ds(off + CHUNK_W, CHUNK_W)],
                bufp[1 - p],
                (psem0, psem1)[1 - p],
            )

        loads = []
        for b in range(4):
            @pl.when(g > 0)
            def _(b=b):
                pltpu.make_async_copy(
                    bx[b], out_hbm.at[pl.ds(0, CHUNK_W)], ssem[b]
                ).wait()
            loads.append(
                pltpu.async_copy(
                    x_hbm.at[pl.ds(b * IMG_W + off, CHUNK_W)], bx[b], lsem[b]
                )
            )

        for cp in loads:
            cp.wait()

        @plsc.parallel_loop(0, VECS, unroll=4)
        def _(i):
            sl = pl.ds(i * 16, 16)
            pv = bufp[p][sl]
            for b in range(4):
                bx[b][sl] = bx[b][sl] + pv

        for b in range(4):
            pltpu.async_copy(
                bx[b], out_hbm.at[pl.ds(b * IMG_W + off, CHUNK_W)], ssem[b]
            )

    def chunk_pair(g2, _):
        do_chunk(g2 * 2, 0)
        do_chunk(g2 * 2 + 1, 1)
        return 0

    lax.fori_loop(0, N_CHUNKS // 2, chunk_pair, 0)

    for b in range(4):
        pltpu.make_async_copy(
            bx[b], out_hbm.at[pl.ds(0, CHUNK_W)], ssem[b]
        ).wait()


def kernel(x, pos_embedding):
    k = pl.kernel(
        _sc_body,
        mesh=plsc.VectorSubcoreMesh(core_axis_name="c", subcore_axis_name="s"),
        out_type=jax.ShapeDtypeStruct((N_BATCH * SEQ * EMB,), jnp.float32),
        scratch_types=(
            [pltpu.VMEM((CHUNK_W,), jnp.float32) for _ in range(6)]
            + [pltpu.SemaphoreType.DMA for _ in range(10)]
        ),
    )
    out = k(x.reshape(-1), pos_embedding.reshape(-1))
    return out.reshape(N_BATCH, SEQ, EMB)


# speedup vs baseline: 3.6937x; 2.4466x over previous
import jax
import jax.numpy as jnp
from jax import lax
from jax.experimental import pallas as pl
from jax.experimental.pallas import tpu as pltpu
from jax.experimental.pallas import tpu_sc as plsc

N_BATCH = 4
SEQ = 4096
EMB = 2048
NC, NS = 2, 16
NW = NC * NS
S_PER_W = SEQ // NW
C = 8
N_CHUNKS = S_PER_W // C
LANE_SL = EMB // 16


def _sc_body(x_hbm, pos_hbm, out_hbm,
             bufp0, bufp1, bx0, bx1, bx2, bx3,
             psem0, psem1, lsem0, lsem1, lsem2, lsem3,
             ssem0, ssem1, ssem2, ssem3):
    wid = lax.axis_index("s") * NC + lax.axis_index("c")
    s_base = wid * S_PER_W

    bufp = (bufp0, bufp1)
    bx = (bx0, bx1, bx2, bx3)
    lsem = (lsem0, lsem1, lsem2, lsem3)
    ssem = (ssem0, ssem1, ssem2, ssem3)

    pltpu.async_copy(pos_hbm.at[pl.ds(s_base, C)], bufp0, psem0)

    def do_chunk(g, p):
        s0 = s_base + g * C

        pltpu.make_async_copy(
            pos_hbm.at[pl.ds(0, C)], bufp[p], (psem0, psem1)[p]
        ).wait()

        @pl.when(g + 1 < N_CHUNKS)
        def _():
            pltpu.async_copy(
                pos_hbm.at[pl.ds(s0 + C, C)], bufp[1 - p], (psem0, psem1)[1 - p]
            )

        loads = []
        for b in range(4):
            @pl.when(g > 0)
            def _(b=b):
                pltpu.make_async_copy(
                    bx[b], out_hbm.at[0].at[pl.ds(0, C)], ssem[b]
                ).wait()
            loads.append(
                pltpu.async_copy(
                    x_hbm.at[b].at[pl.ds(s0, C)], bx[b], lsem[b]
                )
            )

        for cp in loads:
            cp.wait()

        for r in range(C):
            @plsc.parallel_loop(0, LANE_SL, unroll=4)
            def _(i, r=r):
                sl = pl.ds(i * 16, 16)
                pv = bufp[p][r, sl]
                for b in range(4):
                    bx[b][r, sl] = bx[b][r, sl] + pv

        for b in range(4):
            pltpu.async_copy(
                bx[b], out_hbm.at[b].at[pl.ds(s0, C)], ssem[b]
            )

    def chunk_pair(g2, _):
        do_chunk(g2 * 2, 0)
        do_chunk(g2 * 2 + 1, 1)
        return 0

    lax.fori_loop(0, N_CHUNKS // 2, chunk_pair, 0)

    for b in range(4):
        pltpu.make_async_copy(
            bx[b], out_hbm.at[0].at[pl.ds(0, C)], ssem[b]
        ).wait()


def kernel(x, pos_embedding):
    k = pl.kernel(
        _sc_body,
        mesh=plsc.VectorSubcoreMesh(core_axis_name="c", subcore_axis_name="s"),
        out_type=jax.ShapeDtypeStruct((N_BATCH, SEQ, EMB), jnp.float32),
        scratch_types=(
            [pltpu.VMEM((C, EMB), jnp.float32) for _ in range(6)]
            + [pltpu.SemaphoreType.DMA for _ in range(10)]
        ),
    )
    return k(x, pos_embedding)
